# Initial kernel scaffold; baseline (speedup 1.0000x reference)
import jax
import jax.numpy as jnp
from jax.experimental import pallas as pl


def kernel(boxes, scores):
    raise NotImplementedError("write your pallas kernel here")


# speedup vs baseline: 2.9137x; 2.9137x over previous
import jax
import jax.numpy as jnp
from jax.experimental import pallas as pl
from jax.experimental.pallas import tpu as pltpu

NCLS = 20
NREAL = 20000
NPAD = 20480
K = 300
KPAD = 384
NDET = 100
NEG = -3e38
SUPP = -1e9
THRESH = 0.05
IOU_T = 0.5


def _postproc_kernel(s_ref, b_ref, out_ref, s_scr, tv, x1s, y1s, x2s, y2s, kp):
    f32 = jnp.float32
    col = jax.lax.broadcasted_iota(jnp.int32, (NCLS, NPAD), 1)
    kcol = jax.lax.broadcasted_iota(jnp.int32, (NCLS, KPAD), 1)

    s_scr[...] = s_ref[...]
    tv[...] = jnp.full((NCLS, KPAD), NEG, f32)
    x1s[...] = jnp.zeros((NCLS, KPAD), f32)
    y1s[...] = jnp.zeros((NCLS, KPAD), f32)
    x2s[...] = jnp.zeros((NCLS, KPAD), f32)
    y2s[...] = jnp.zeros((NCLS, KPAD), f32)

    def body1(k, _):
        s = s_scr[...]
        m = jnp.max(s, axis=1)
        eq = s == m[:, None]
        idx = jnp.min(jnp.where(eq, col, 2 ** 30), axis=1)
        sel = col == idx[:, None]
        s_scr[...] = jnp.where(sel, NEG, s)
        self = sel.astype(f32)
        km = kcol == k
        tv[...] = jnp.where(km, m[:, None], tv[...])
        x1 = jnp.sum(b_ref[0] * self, axis=1)
        y1 = jnp.sum(b_ref[1] * self, axis=1)
        x2 = jnp.sum(b_ref[2] * self, axis=1)
        y2 = jnp.sum(b_ref[3] * self, axis=1)
        x1s[...] = jnp.where(km, x1[:, None], x1s[...])
        y1s[...] = jnp.where(km, y1[:, None], y1s[...])
        x2s[...] = jnp.where(km, x2[:, None], x2s[...])
        y2s[...] = jnp.where(km, y2[:, None], y2s[...])
        return 0

    jax.lax.fori_loop(0, K, body1, 0)

    bx1 = x1s[...]
    by1 = y1s[...]
    bx2 = x2s[...]
    by2 = y2s[...]
    tvv = tv[...]
    area = jnp.maximum(bx2 - bx1, 0.0) * jnp.maximum(by2 - by1, 0.0)
    kp[...] = (tvv > THRESH).astype(f32)

    def body2(i, _):
        kf = kp[...]
        im = (kcol == i).astype(f32)
        x1i = jnp.sum(bx1 * im, axis=1)
        y1i = jnp.sum(by1 * im, axis=1)
        x2i = jnp.sum(bx2 * im, axis=1)
        y2i = jnp.sum(by2 * im, axis=1)
        ai = jnp.sum(area * im, axis=1)
        ki = jnp.sum(kf * im, axis=1)
        iw = jnp.maximum(jnp.minimum(bx2, x2i[:, None]) - jnp.maximum(bx1, x1i[:, None]), 0.0)
        ih = jnp.maximum(jnp.minimum(by2, y2i[:, None]) - jnp.maximum(by1, y1i[:, None]), 0.0)
        inter = iw * ih
        union = ai[:, None] + area - inter
        iou = inter / jnp.maximum(union, 1e-9)
        sup = (iou > IOU_T) & (ki[:, None] > 0.5) & (kcol > i)
        kp[...] = kf * (1.0 - sup.astype(f32))
        return 0

    jax.lax.fori_loop(0, K, body2, 0)

    ss0 = jnp.where(kcol < K, jnp.where(kp[...] > 0.5, tvv, SUPP), NEG)
    ccls = jax.lax.broadcasted_iota(jnp.int32, (NCLS, KPAD), 0)
    flat = ccls * KPAD + kcol
    row8 = jax.lax.broadcasted_iota(jnp.int32, (8, 128), 0)
    col128 = jax.lax.broadcasted_iota(jnp.int32, (8, 128), 1)
    out_ref[...] = jnp.zeros((8, 128), f32)

    def body3(k, ssv):
        m = jnp.max(ssv)
        eq = ssv == m
        fi = jnp.min(jnp.where(eq, flat, 2 ** 30))
        sel = flat == fi
        self = sel.astype(f32)
        x1 = jnp.sum(bx1 * self)
        y1 = jnp.sum(by1 * self)
        x2 = jnp.sum(bx2 * self)
        y2 = jnp.sum(by2 * self)
        lab = (fi // KPAD + 1).astype(f32)
        val = jnp.where(row8 == 0, x1,
              jnp.where(row8 == 1, y1,
              jnp.where(row8 == 2, x2,
              jnp.where(row8 == 3, y2,
              jnp.where(row8 == 4, m,
              jnp.where(row8 == 5, lab, 0.0))))))
        out_ref[...] = jnp.where(col128 == k, val, out_ref[...])
        return jnp.where(sel, NEG, ssv)

    jax.lax.fori_loop(0, NDET, body3, ss0)


def kernel(boxes, scores):
    s = scores[:, 1:].T
    s = jnp.pad(s, ((0, 0), (0, NPAD - NREAL)), constant_values=NEG)
    b = boxes.reshape(NREAL, NCLS + 1, 4)[:, 1:, :].transpose(2, 1, 0)
    b = jnp.pad(b, ((0, 0), (0, 0), (0, NPAD - NREAL)))
    out = pl.pallas_call(
        _postproc_kernel,
        out_shape=jax.ShapeDtypeStruct((8, 128), jnp.float32),
        scratch_shapes=[
            pltpu.VMEM((NCLS, NPAD), jnp.float32),
            pltpu.VMEM((NCLS, KPAD), jnp.float32),
            pltpu.VMEM((NCLS, KPAD), jnp.float32),
            pltpu.VMEM((NCLS, KPAD), jnp.float32),
            pltpu.VMEM((NCLS, KPAD), jnp.float32),
            pltpu.VMEM((NCLS, KPAD), jnp.float32),
            pltpu.VMEM((NCLS, KPAD), jnp.float32),
        ],
    )(s, b)
    det = jnp.stack([out[0, :NDET], out[1, :NDET], out[2, :NDET],
                     out[3, :NDET], out[4, :NDET]], axis=1)
    labels = out[5, :NDET].astype(jnp.int32)
    return det, labels
